# baseline (speedup 1.0000x reference)
import numpy as np
import jax
import jax.numpy as jnp
from jax.experimental import pallas as pl
from jax.experimental.pallas import tpu as pltpu

_SLOW_FRAMES = 8
_NBUF = 16
_LOOKAHEAD = 8


def _make_body(idx, C, T):
    slot_of = {t: j for j, t in enumerate(idx)}
    tasks = [(c, t) for t in range(T) for c in range(C)]
    n = len(tasks)

    def _body(frames_ref, slow_ref, fast_ref, buf, rsem, fsem, ssem):
        reads, fwrites, swrites = {}, {}, {}
        for k, (c, t) in enumerate(tasks):
            b = k % _NBUF
            reads[k] = pltpu.make_async_copy(
                frames_ref.at[c:c + 1, t:t + 1], buf.at[b], rsem.at[b]
            )
            fwrites[k] = pltpu.make_async_copy(
                buf.at[b], fast_ref.at[c:c + 1, t:t + 1], fsem.at[b]
            )
            if t in slot_of:
                j = slot_of[t]
                swrites[k] = pltpu.make_async_copy(
                    buf.at[b], slow_ref.at[c:c + 1, j:j + 1], ssem.at[b]
                )

        for step in range(n + _LOOKAHEAD):
            k = step
            if k < n:
                if k >= _NBUF:
                    fwrites[k - _NBUF].wait()
                    if (k - _NBUF) in swrites:
                        swrites[k - _NBUF].wait()
                reads[k].start()
            u = step - _LOOKAHEAD
            if u >= 0:
                reads[u].wait()
                fwrites[u].start()
                if u in swrites:
                    swrites[u].start()
        for k in range(n - _NBUF, n):
            fwrites[k].wait()
            if k in swrites:
                swrites[k].wait()

    return _body


def kernel(frames):
    C, T, H, W = frames.shape
    idx = [int(v) for v in np.linspace(0.0, float(T - 1), _SLOW_FRAMES).astype(np.int32)]

    slow, fast = pl.pallas_call(
        _make_body(idx, C, T),
        in_specs=[pl.BlockSpec(memory_space=pltpu.MemorySpace.HBM)],
        out_specs=(
            pl.BlockSpec(memory_space=pltpu.MemorySpace.HBM),
            pl.BlockSpec(memory_space=pltpu.MemorySpace.HBM),
        ),
        out_shape=(
            jax.ShapeDtypeStruct((C, _SLOW_FRAMES, H, W), frames.dtype),
            jax.ShapeDtypeStruct((C, T, H, W), frames.dtype),
        ),
        scratch_shapes=[
            pltpu.VMEM((_NBUF, 1, 1, H, W), frames.dtype),
            pltpu.SemaphoreType.DMA((_NBUF,)),
            pltpu.SemaphoreType.DMA((_NBUF,)),
            pltpu.SemaphoreType.DMA((_NBUF,)),
        ],
    )(frames)
    return (slow, fast)


# speedup vs baseline: 1.2328x; 1.2328x over previous
import numpy as np
import jax
import jax.numpy as jnp
from jax.experimental import pallas as pl
from jax.experimental.pallas import tpu as pltpu

_SLOW_FRAMES = 8
_CHUNK = 2
_NBUF = 8
_LOOKAHEAD = 4


def _make_body(idx, T):
    slot_of = {t: j for j, t in enumerate(idx)}
    n = T // _CHUNK

    def _body(frames_ref, slow_ref, fast_ref, buf, rsem, fsem, ssem):
        reads, fwrites, swrites = {}, {}, {}
        for k in range(n):
            b = k % _NBUF
            t0 = k * _CHUNK
            reads[k] = pltpu.make_async_copy(
                frames_ref.at[:, t0:t0 + _CHUNK], buf.at[b], rsem.at[b]
            )
            fwrites[k] = pltpu.make_async_copy(
                buf.at[b], fast_ref.at[:, t0:t0 + _CHUNK], fsem.at[b]
            )
            sw = []
            for o in range(_CHUNK):
                t = t0 + o
                if t in slot_of:
                    j = slot_of[t]
                    sw.append(
                        pltpu.make_async_copy(
                            buf.at[b, :, o:o + 1],
                            slow_ref.at[:, j:j + 1],
                            ssem.at[b],
                        )
                    )
            if sw:
                swrites[k] = sw

        for step in range(n + _LOOKAHEAD):
            k = step
            if k < n:
                if k >= _NBUF:
                    fwrites[k - _NBUF].wait()
                    for c in swrites.get(k - _NBUF, ()):
                        c.wait()
                reads[k].start()
            u = step - _LOOKAHEAD
            if u >= 0:
                reads[u].wait()
                fwrites[u].start()
                for c in swrites.get(u, ()):
                    c.start()
        for k in range(max(0, n - _NBUF), n):
            fwrites[k].wait()
            for c in swrites.get(k, ()):
                c.wait()

    return _body


def kernel(frames):
    C, T, H, W = frames.shape
    idx = [int(v) for v in np.linspace(0.0, float(T - 1), _SLOW_FRAMES).astype(np.int32)]

    slow, fast = pl.pallas_call(
        _make_body(idx, T),
        in_specs=[pl.BlockSpec(memory_space=pltpu.MemorySpace.HBM)],
        out_specs=(
            pl.BlockSpec(memory_space=pltpu.MemorySpace.HBM),
            pl.BlockSpec(memory_space=pltpu.MemorySpace.HBM),
        ),
        out_shape=(
            jax.ShapeDtypeStruct((C, _SLOW_FRAMES, H, W), frames.dtype),
            jax.ShapeDtypeStruct((C, T, H, W), frames.dtype),
        ),
        scratch_shapes=[
            pltpu.VMEM((_NBUF, C, _CHUNK, H, W), frames.dtype),
            pltpu.SemaphoreType.DMA((_NBUF,)),
            pltpu.SemaphoreType.DMA((_NBUF,)),
            pltpu.SemaphoreType.DMA((_NBUF,)),
        ],
    )(frames)
    return (slow, fast)


# speedup vs baseline: 1.2588x; 1.0211x over previous
import numpy as np
import jax
import jax.numpy as jnp
from jax.experimental import pallas as pl
from jax.experimental.pallas import tpu as pltpu

_SLOW_FRAMES = 8
_CHUNK = 4
_NBUF = 6
_LOOKAHEAD = 3


def _make_body(idx, T):
    slot_of = {t: j for j, t in enumerate(idx)}
    n = T // _CHUNK

    def _body(frames_ref, slow_ref, fast_ref, buf, rsem, fsem, ssem):
        reads, fwrites, swrites = {}, {}, {}
        for k in range(n):
            b = k % _NBUF
            t0 = k * _CHUNK
            reads[k] = pltpu.make_async_copy(
                frames_ref.at[:, t0:t0 + _CHUNK], buf.at[b], rsem.at[b]
            )
            fwrites[k] = pltpu.make_async_copy(
                buf.at[b], fast_ref.at[:, t0:t0 + _CHUNK], fsem.at[b]
            )
            sw = []
            for o in range(_CHUNK):
                t = t0 + o
                if t in slot_of:
                    j = slot_of[t]
                    sw.append(
                        pltpu.make_async_copy(
                            buf.at[b, :, o:o + 1],
                            slow_ref.at[:, j:j + 1],
                            ssem.at[b],
                        )
                    )
            if sw:
                swrites[k] = sw

        for step in range(n + _LOOKAHEAD):
            k = step
            if k < n:
                if k >= _NBUF:
                    fwrites[k - _NBUF].wait()
                    for c in swrites.get(k - _NBUF, ()):
                        c.wait()
                reads[k].start()
            u = step - _LOOKAHEAD
            if u >= 0:
                reads[u].wait()
                fwrites[u].start()
                for c in swrites.get(u, ()):
                    c.start()
        for k in range(max(0, n - _NBUF), n):
            fwrites[k].wait()
            for c in swrites.get(k, ()):
                c.wait()

    return _body


def kernel(frames):
    C, T, H, W = frames.shape
    idx = [int(v) for v in np.linspace(0.0, float(T - 1), _SLOW_FRAMES).astype(np.int32)]

    slow, fast = pl.pallas_call(
        _make_body(idx, T),
        in_specs=[pl.BlockSpec(memory_space=pltpu.MemorySpace.HBM)],
        out_specs=(
            pl.BlockSpec(memory_space=pltpu.MemorySpace.HBM),
            pl.BlockSpec(memory_space=pltpu.MemorySpace.HBM),
        ),
        out_shape=(
            jax.ShapeDtypeStruct((C, _SLOW_FRAMES, H, W), frames.dtype),
            jax.ShapeDtypeStruct((C, T, H, W), frames.dtype),
        ),
        scratch_shapes=[
            pltpu.VMEM((_NBUF, C, _CHUNK, H, W), frames.dtype),
            pltpu.SemaphoreType.DMA((_NBUF,)),
            pltpu.SemaphoreType.DMA((_NBUF,)),
            pltpu.SemaphoreType.DMA((_NBUF,)),
        ],
    )(frames)
    return (slow, fast)


# speedup vs baseline: 1.2682x; 1.0075x over previous
import numpy as np
import jax
import jax.numpy as jnp
from jax.experimental import pallas as pl
from jax.experimental.pallas import tpu as pltpu

_SLOW_FRAMES = 8
_CHUNK = 8
_NBUF = 4
_LOOKAHEAD = 2


def _make_body(idx, T):
    slot_of = {t: j for j, t in enumerate(idx)}
    n = T // _CHUNK

    def _body(frames_ref, slow_ref, fast_ref, buf, rsem, fsem, ssem):
        reads, fwrites, swrites = {}, {}, {}
        for k in range(n):
            b = k % _NBUF
            t0 = k * _CHUNK
            reads[k] = pltpu.make_async_copy(
                frames_ref.at[:, t0:t0 + _CHUNK], buf.at[b], rsem.at[b]
            )
            fwrites[k] = pltpu.make_async_copy(
                buf.at[b], fast_ref.at[:, t0:t0 + _CHUNK], fsem.at[b]
            )
            sw = []
            for o in range(_CHUNK):
                t = t0 + o
                if t in slot_of:
                    j = slot_of[t]
                    sw.append(
                        pltpu.make_async_copy(
                            buf.at[b, :, o:o + 1],
                            slow_ref.at[:, j:j + 1],
                            ssem.at[b],
                        )
                    )
            if sw:
                swrites[k] = sw

        for step in range(n + _LOOKAHEAD):
            k = step
            if k < n:
                if k >= _NBUF:
                    fwrites[k - _NBUF].wait()
                    for c in swrites.get(k - _NBUF, ()):
                        c.wait()
                reads[k].start()
            u = step - _LOOKAHEAD
            if u >= 0:
                reads[u].wait()
                fwrites[u].start()
                for c in swrites.get(u, ()):
                    c.start()
        for k in range(max(0, n - _NBUF), n):
            fwrites[k].wait()
            for c in swrites.get(k, ()):
                c.wait()

    return _body


def kernel(frames):
    C, T, H, W = frames.shape
    idx = [int(v) for v in np.linspace(0.0, float(T - 1), _SLOW_FRAMES).astype(np.int32)]

    slow, fast = pl.pallas_call(
        _make_body(idx, T),
        in_specs=[pl.BlockSpec(memory_space=pltpu.MemorySpace.HBM)],
        out_specs=(
            pl.BlockSpec(memory_space=pltpu.MemorySpace.HBM),
            pl.BlockSpec(memory_space=pltpu.MemorySpace.HBM),
        ),
        out_shape=(
            jax.ShapeDtypeStruct((C, _SLOW_FRAMES, H, W), frames.dtype),
            jax.ShapeDtypeStruct((C, T, H, W), frames.dtype),
        ),
        scratch_shapes=[
            pltpu.VMEM((_NBUF, C, _CHUNK, H, W), frames.dtype),
            pltpu.SemaphoreType.DMA((_NBUF,)),
            pltpu.SemaphoreType.DMA((_NBUF,)),
            pltpu.SemaphoreType.DMA((_NBUF,)),
        ],
    )(frames)
    return (slow, fast)
